# baseline (speedup 1.0000x reference)
import functools

import jax
import jax.numpy as jnp
from jax import lax
from jax.experimental import pallas as pl
from jax.experimental.pallas import tpu as pltpu
from jax.experimental.pallas import tpu_sc as plsc

N = 10000
E = 320000
C = 128
DELTA = 3.5

NC, NS, L = 2, 16, 16
NW = NC * NS
NR = 2 * NW
NPR = 160
NTOT = NR * NPR
CH = 512
FB = 128
OB = 640
ECAP = E + FB
NFLUSH = 4

_mesh = plsc.VectorSubcoreMesh(core_axis_name="c", subcore_axis_name="s")
_params = pltpu.CompilerParams(needs_layout_passes=False)


def _wid():
    return lax.axis_index("s") * NC + lax.axis_index("c")


def _al(x):
    return pl.multiple_of(x, 8)


def _sload(ref, i):
    return ref[pl.ds(i, L)][0]


@functools.partial(
    pl.kernel,
    out_type=(
        jax.ShapeDtypeStruct((NR * ECAP,), jnp.int32),
        jax.ShapeDtypeStruct((NR * ECAP,), jnp.int32),
        jax.ShapeDtypeStruct((NR * L,), jnp.int32),
    ),
    mesh=_mesh,
    compiler_params=_params,
    scratch_types=[
        pltpu.VMEM((CH,), jnp.int32),
        pltpu.VMEM((CH,), jnp.int32),
        pltpu.VMEM((CH,), jnp.int32),
        pltpu.VMEM((CH,), jnp.int32),
        pltpu.VMEM((OB,), jnp.int32),
        pltpu.VMEM((OB,), jnp.int32),
        pltpu.VMEM((OB,), jnp.int32),
        pltpu.VMEM((OB,), jnp.int32),
        pltpu.VMEM((L,), jnp.int32),
        pltpu.SMEM((8,), jnp.int32),
        pltpu.SemaphoreType.DMA,
        pltpu.SemaphoreType.DMA,
    ],
)
def _phase_a(src_hbm, dst_hbm, srcs_out, offs_out, cnts_out,
             dbuf0, dbuf1, sbuf0, sbuf1, stg_s0, stg_o0, stg_s1, stg_o1,
             cbuf, kref, sem0, sem1):
    wid = _wid()
    base0 = (2 * wid) * NPR
    stg = ((stg_s0, stg_o0), (stg_s1, stg_o1))
    bufs = ((dbuf0, sbuf0, sem0), (dbuf1, sbuf1, sem1))
    NCHUNK = E // CH

    kref[0] = 0
    kref[1] = 0
    kref[2] = 0
    kref[3] = 0

    def flush(r, pred):
        stg_s, stg_o = stg[r]
        ebase = (2 * wid + r) * ECAP

        @pl.when(pred)
        def _():
            ptr = kref[2 * r + 1]
            pltpu.sync_copy(stg_s.at[pl.ds(0, FB)],
                            srcs_out.at[pl.ds(_al(ebase + ptr), FB)])
            pltpu.sync_copy(stg_o.at[pl.ds(0, FB)],
                            offs_out.at[pl.ds(_al(ebase + ptr), FB)])

            def mv(i, _):
                stg_s[pl.ds(i * L, L)] = stg_s[pl.ds(i * L + FB, L)]
                stg_o[pl.ds(i * L, L)] = stg_o[pl.ds(i * L + FB, L)]
                return 0

            lax.fori_loop(0, (OB - FB) // L, mv, 0)
            kref[2 * r] = kref[2 * r] - FB
            kref[2 * r + 1] = ptr + FB

    def start(c, dbuf, sbuf, sem):
        pltpu.async_copy(dst_hbm.at[pl.ds(c * CH, CH)], dbuf, sem)
        pltpu.async_copy(src_hbm.at[pl.ds(c * CH, CH)], sbuf, sem)

    def consume(dbuf, sbuf, sem):
        pltpu.make_async_copy(dst_hbm.at[pl.ds(0, CH)], dbuf, sem).wait()
        pltpu.make_async_copy(src_hbm.at[pl.ds(0, CH)], sbuf, sem).wait()

        def sub(i, kv):
            kv0, kv1 = kv
            d16 = dbuf[pl.ds(i * L, L)]
            s16 = sbuf[pl.ds(i * L, L)]
            off = d16 - base0
            m0 = (off >= 0) & (off < NPR)
            m1 = (off >= NPR) & (off < 2 * NPR)
            pos0 = kv0 + plsc.cumsum(m0.astype(jnp.int32)) - 1
            plsc.store_scatter(stg_s0, [pos0], s16, mask=m0)
            plsc.store_scatter(stg_o0, [pos0], off, mask=m0)
            pos1 = kv1 + plsc.cumsum(m1.astype(jnp.int32)) - 1
            plsc.store_scatter(stg_s1, [pos1], s16, mask=m1)
            plsc.store_scatter(stg_o1, [pos1], off - NPR, mask=m1)
            return (kv0 + plsc.all_reduce_population_count(m0),
                    kv1 + plsc.all_reduce_population_count(m1))

        k0 = jnp.full((L,), kref[0], jnp.int32)
        k1 = jnp.full((L,), kref[2], jnp.int32)
        kv0, kv1 = lax.fori_loop(0, CH // L, sub, (k0, k1))
        kref[0] = jnp.max(kv0)
        kref[2] = jnp.max(kv1)
        for r in range(2):
            for _ in range(NFLUSH):
                flush(r, kref[2 * r] >= FB)

    for p in range(2):
        if p < NCHUNK:
            start(p, *bufs[p])

    def pair(i2, _):
        for p in range(2):
            b = 2 * i2 + p
            dbuf, sbuf, sem = bufs[p]

            @pl.when(b < NCHUNK)
            def _(b=b, dbuf=dbuf, sbuf=sbuf, sem=sem):
                consume(dbuf, sbuf, sem)

                @pl.when(b + 2 < NCHUNK)
                def _():
                    start(b + 2, dbuf, sbuf, sem)
        return 0

    lax.fori_loop(0, (NCHUNK + 1) // 2, pair, 0)

    zeros = jnp.zeros((L,), jnp.int32)
    trash = jnp.full((L,), NPR, jnp.int32)
    iota = lax.iota(jnp.int32, L)
    for r in range(2):
        g = 2 * wid + r
        stg_s, stg_o = stg[r]
        k = kref[2 * r]
        cnt = kref[2 * r + 1] + k
        for i in range(FB // L):
            pos = k + i * L + iota
            plsc.store_scatter(stg_s, [pos], zeros)
            plsc.store_scatter(stg_o, [pos], trash)
        kref[2 * r] = k + FB
        flush(r, jnp.bool_(True))
        cbuf[...] = jnp.full((L,), cnt, jnp.int32)
        pltpu.sync_copy(cbuf, cnts_out.at[pl.ds(_al(g * L), L)])


def _make_sc_layer(with_deg):
    outs = [jax.ShapeDtypeStruct((NTOT * C,), jnp.float32)] * 4
    scratch = [
        pltpu.VMEM(((NPR + 1) * C,), jnp.float32),
        pltpu.VMEM(((NPR + 1) * C,), jnp.float32),
        pltpu.VMEM(((NPR + 1) * C,), jnp.float32),
        pltpu.VMEM(((NPR + 1) * C,), jnp.float32),
        pltpu.VMEM((FB, C), jnp.float32),
        pltpu.VMEM((FB, C), jnp.float32),
        pltpu.VMEM((FB,), jnp.int32),
        pltpu.VMEM((FB,), jnp.int32),
        pltpu.VMEM((FB + L,), jnp.int32),
        pltpu.VMEM((FB + L,), jnp.int32),
        pltpu.VMEM((L,), jnp.int32),
        pltpu.SemaphoreType.DMA,
        pltpu.SemaphoreType.DMA,
    ]
    if with_deg:
        outs = outs + [jax.ShapeDtypeStruct((NTOT * L,), jnp.float32)]
        scratch = scratch + [pltpu.VMEM(((NPR + 1) * L,), jnp.float32)]

    def body(a_hbm, srcs, offs, cnts, s1_o, s2_o, mx_o, mn_o, *rest):
        if with_deg:
            deg_o = rest[0]
            rest = rest[1:]
            accD = rest[-1]
            rest = rest[:-1]
        (accS, accQ, accMX, accMN, rows0, rows1, sbuf0, sbuf1,
         obuf0, obuf1, cbuf, sem0, sem1) = rest
        wid = _wid()
        zero = jnp.zeros((L,), jnp.float32)
        neg = jnp.full((L,), -3.4e38, jnp.float32)
        pos = jnp.full((L,), 3.4e38, jnp.float32)
        ones = jnp.ones((L,), jnp.float32)
        bufs = ((rows0, sbuf0, obuf0, sem0), (rows1, sbuf1, obuf1, sem1))

        for r in range(2):
            g = 2 * wid + r
            ebase = g * ECAP
            pltpu.sync_copy(cnts.at[pl.ds(_al(g * L), L)], cbuf)
            cnt = jnp.max(cbuf[...])
            nblk = (cnt + FB - 1) // FB

            def initrow(i, _):
                for j in range(C // L):
                    sl = pl.ds(i * C + j * L, L)
                    accS[sl] = zero
                    accQ[sl] = zero
                    accMX[sl] = neg
                    accMN[sl] = pos
                if with_deg:
                    accD[pl.ds(i * L, L)] = zero
                return 0

            lax.fori_loop(0, NPR, initrow, 0)

            def start(b, rows, sbuf, obuf, sem):
                pltpu.sync_copy(srcs.at[pl.ds(_al(ebase + b * FB), FB)], sbuf)
                pltpu.sync_copy(offs.at[pl.ds(_al(ebase + b * FB), FB)],
                                obuf.at[pl.ds(0, FB)])
                pltpu.async_copy(a_hbm.at[sbuf], rows, sem)

            def consume(rows, sbuf, obuf, sem):
                pltpu.make_async_copy(a_hbm.at[sbuf], rows, sem).wait()

                def edge(e, _):
                    off = _sload(obuf, e)
                    for j in range(C // L):
                        sl = pl.ds(off * C + j * L, L)
                        x = rows[e, pl.ds(j * L, L)]
                        plsc.addupdate(accS.at[sl], x)
                        plsc.addupdate(accQ.at[sl], x * x)
                        accMX[sl] = jnp.maximum(accMX[sl], x)
                        accMN[sl] = jnp.minimum(accMN[sl], x)
                    if with_deg:
                        plsc.addupdate(accD.at[pl.ds(off * L, L)], ones)
                    return 0

                lax.fori_loop(0, FB, edge, 0, unroll=2)

            for p in range(2):
                @pl.when(p < nblk)
                def _(p=p):
                    rows, sbuf, obuf, sem = bufs[p]
                    start(p, rows, sbuf, obuf, sem)

            def pair(i2, _):
                for p in range(2):
                    b = 2 * i2 + p
                    rows, sbuf, obuf, sem = bufs[p]

                    @pl.when(b < nblk)
                    def _(b=b, rows=rows, sbuf=sbuf, obuf=obuf, sem=sem):
                        consume(rows, sbuf, obuf, sem)

                        @pl.when(b + 2 < nblk)
                        def _():
                            start(b + 2, rows, sbuf, obuf, sem)
                return 0

            lax.fori_loop(0, (nblk + 1) // 2, pair, 0)

            hsl = pl.ds(_al(g * (NPR * C)), NPR * C)
            pltpu.sync_copy(accS.at[pl.ds(0, NPR * C)], s1_o.at[hsl])
            pltpu.sync_copy(accQ.at[pl.ds(0, NPR * C)], s2_o.at[hsl])
            pltpu.sync_copy(accMX.at[pl.ds(0, NPR * C)], mx_o.at[hsl])
            pltpu.sync_copy(accMN.at[pl.ds(0, NPR * C)], mn_o.at[hsl])
            if with_deg:
                pltpu.sync_copy(
                    accD.at[pl.ds(0, NPR * L)],
                    deg_o.at[pl.ds(_al(g * (NPR * L)), NPR * L)])

    return pl.kernel(
        body,
        out_type=tuple(outs),
        mesh=_mesh,
        compiler_params=_params,
        scratch_types=scratch,
    )


_sc_layer0 = _make_sc_layer(True)
_sc_layer = _make_sc_layer(False)


BLK = 400


def _pre_body(h_ref, w_ref, b_ref, a_ref, bout_ref):
    h = h_ref[...]
    a_ref[...] = jnp.dot(h, w_ref[0:C, :], preferred_element_type=jnp.float32)
    bout_ref[...] = (jnp.dot(h, w_ref[C:2 * C, :],
                             preferred_element_type=jnp.float32)
                     + b_ref[...])


_pre = pl.pallas_call(
    _pre_body,
    grid=(N // BLK,),
    in_specs=[
        pl.BlockSpec((BLK, C), lambda i: (i, 0)),
        pl.BlockSpec((2 * C, C), lambda i: (0, 0)),
        pl.BlockSpec((1, C), lambda i: (0, 0)),
    ],
    out_specs=[
        pl.BlockSpec((BLK, C), lambda i: (i, 0)),
        pl.BlockSpec((BLK, C), lambda i: (i, 0)),
    ],
    out_shape=[
        jax.ShapeDtypeStruct((N, C), jnp.float32),
        jax.ShapeDtypeStruct((N, C), jnp.float32),
    ],
)


def _post_body(residual, h_ref, b_ref, s1_ref, s2_ref, mx_ref, mn_ref,
               deg_ref, uw_ref, ub_ref, mw_ref, mb_ref, o_ref):
    h = h_ref[...]
    B = b_ref[...]
    d = deg_ref[:, 0:1]
    cnt = jnp.maximum(d, 1.0)
    has = d > 0
    S1 = s1_ref[...]
    mean = (S1 + d * B) / cnt
    msq = (s2_ref[...] + 2.0 * B * S1 + d * B * B) / cnt
    std = jnp.sqrt(jnp.maximum(msq - mean * mean, 0.0) + 1e-5)
    mx = jnp.where(has, mx_ref[...] + B, 0.0)
    mn = jnp.where(has, mn_ref[...] + B, 0.0)
    aggs = jnp.concatenate([mean, mx, mn, std], axis=1)
    logd = jnp.log(d + 1.0)
    amp = logd / DELTA
    att = jnp.where(logd > 0, DELTA / jnp.where(logd > 0, logd, 1.0), 0.0)
    uw = uw_ref[...]
    u = (jnp.dot(h, uw[0:C, :], preferred_element_type=jnp.float32)
         + jnp.dot(aggs, uw[C:C + 512, :], preferred_element_type=jnp.float32)
         + jnp.dot(aggs * amp, uw[C + 512:C + 1024, :],
                   preferred_element_type=jnp.float32)
         + jnp.dot(aggs * att, uw[C + 1024:C + 1536, :],
                   preferred_element_type=jnp.float32)
         + ub_ref[...])
    o = jnp.dot(u, mw_ref[...], preferred_element_type=jnp.float32) + mb_ref[...]
    o = jnp.where(o >= 0, o, 0.01 * o)
    if residual:
        o = o + h
    o_ref[...] = o


@functools.cache
def _post(cout, residual):
    uin = 13 * C
    full_spec = pl.BlockSpec((BLK, C), lambda i: (i, 0))
    return pl.pallas_call(
        functools.partial(_post_body, residual),
        grid=(N // BLK,),
        in_specs=[
            full_spec,
            full_spec,
            full_spec, full_spec, full_spec, full_spec,
            pl.BlockSpec((BLK, L), lambda i: (i, 0)),
            pl.BlockSpec((uin, cout), lambda i: (0, 0)),
            pl.BlockSpec((1, cout), lambda i: (0, 0)),
            pl.BlockSpec((cout, cout), lambda i: (0, 0)),
            pl.BlockSpec((1, cout), lambda i: (0, 0)),
        ],
        out_specs=pl.BlockSpec((BLK, cout), lambda i: (i, 0)),
        out_shape=jax.ShapeDtypeStruct((N, cout), jnp.float32),
    )


def _mean_body(x_ref, o_ref):
    o_ref[...] = jnp.mean(x_ref[...], axis=0, keepdims=True)


def _mean(x):
    n, c = x.shape
    return pl.pallas_call(
        _mean_body,
        out_shape=jax.ShapeDtypeStruct((1, c), jnp.float32),
    )(x)


def kernel(features, edge_index,
           M_W0, M_b0, U_W0, U_b0, mix_W0, mix_b0,
           M_W1, M_b1, U_W1, U_b1, mix_W1, mix_b1,
           M_W2, M_b2, U_W2, U_b2, mix_W2, mix_b2):
    src = edge_index[0]
    dst = edge_index[1]
    srcs, offs, cnts = _phase_a(src, dst)

    params = [
        (M_W0, M_b0, U_W0, U_b0, mix_W0, mix_b0),
        (M_W1, M_b1, U_W1, U_b1, mix_W1, mix_b1),
        (M_W2, M_b2, U_W2, U_b2, mix_W2, mix_b2),
    ]
    h = features
    deg = None
    for l, (MW, Mb, UW, Ub, mixW, mixb) in enumerate(params):
        a, B = _pre(h, MW, Mb.reshape(1, C))
        if l == 0:
            s1, s2, mx, mn, degf = _sc_layer0(a, srcs, offs, cnts)
            deg = degf.reshape(NTOT, L)[0:N]
            stats = (s1, s2, mx, mn)
        else:
            stats = _sc_layer(a, srcs, offs, cnts)
        s1, s2, mx, mn = (s.reshape(NTOT, C)[0:N] for s in stats)
        cout = UW.shape[1]
        h = _post(cout, l < 2)(h, B, s1, s2, mx, mn,
                               deg, UW, Ub.reshape(1, cout), mixW,
                               mixb.reshape(1, cout))
    return _mean(h)


# speedup vs baseline: 1.1236x; 1.1236x over previous
import functools

import jax
import jax.numpy as jnp
from jax import lax
from jax.experimental import pallas as pl
from jax.experimental.pallas import tpu as pltpu
from jax.experimental.pallas import tpu_sc as plsc

N = 10000
E = 320000
C = 128
DELTA = 3.5

NC, NS, L = 2, 16, 16
NW = NC * NS
NR = 2 * NW
NPR = 160
NTOT = NR * NPR
CH = 512
FB = 128
OB = 640
ECAP = E + FB
NFLUSH = 4

_mesh = plsc.VectorSubcoreMesh(core_axis_name="c", subcore_axis_name="s")
_params = pltpu.CompilerParams(needs_layout_passes=False)


def _wid():
    return lax.axis_index("s") * NC + lax.axis_index("c")


def _al(x):
    return pl.multiple_of(x, 8)


def _sload(ref, i):
    return ref[pl.ds(i, L)][0]


@functools.partial(
    pl.kernel,
    out_type=(
        jax.ShapeDtypeStruct((NR * ECAP,), jnp.int32),
        jax.ShapeDtypeStruct((NR * ECAP,), jnp.int32),
        jax.ShapeDtypeStruct((NR * L,), jnp.int32),
    ),
    mesh=_mesh,
    compiler_params=_params,
    scratch_types=[
        pltpu.VMEM((CH,), jnp.int32),
        pltpu.VMEM((CH,), jnp.int32),
        pltpu.VMEM((CH,), jnp.int32),
        pltpu.VMEM((CH,), jnp.int32),
        pltpu.VMEM((OB,), jnp.int32),
        pltpu.VMEM((OB,), jnp.int32),
        pltpu.VMEM((OB,), jnp.int32),
        pltpu.VMEM((OB,), jnp.int32),
        pltpu.VMEM((L,), jnp.int32),
        pltpu.SMEM((8,), jnp.int32),
        pltpu.SemaphoreType.DMA,
        pltpu.SemaphoreType.DMA,
    ],
)
def _phase_a(src_hbm, dst_hbm, srcs_out, offs_out, cnts_out,
             dbuf0, dbuf1, sbuf0, sbuf1, stg_s0, stg_o0, stg_s1, stg_o1,
             cbuf, kref, sem0, sem1):
    wid = _wid()
    base0 = (2 * wid) * NPR
    stg = ((stg_s0, stg_o0), (stg_s1, stg_o1))
    bufs = ((dbuf0, sbuf0, sem0), (dbuf1, sbuf1, sem1))
    NCHUNK = E // CH

    kref[0] = 0
    kref[1] = 0
    kref[2] = 0
    kref[3] = 0

    def flush(r, pred):
        stg_s, stg_o = stg[r]
        ebase = (2 * wid + r) * ECAP

        @pl.when(pred)
        def _():
            ptr = kref[2 * r + 1]
            pltpu.sync_copy(stg_s.at[pl.ds(0, FB)],
                            srcs_out.at[pl.ds(_al(ebase + ptr), FB)])
            pltpu.sync_copy(stg_o.at[pl.ds(0, FB)],
                            offs_out.at[pl.ds(_al(ebase + ptr), FB)])

            def mv(i, _):
                stg_s[pl.ds(i * L, L)] = stg_s[pl.ds(i * L + FB, L)]
                stg_o[pl.ds(i * L, L)] = stg_o[pl.ds(i * L + FB, L)]
                return 0

            lax.fori_loop(0, (OB - FB) // L, mv, 0)
            kref[2 * r] = kref[2 * r] - FB
            kref[2 * r + 1] = ptr + FB

    def start(c, dbuf, sbuf, sem):
        pltpu.async_copy(dst_hbm.at[pl.ds(c * CH, CH)], dbuf, sem)
        pltpu.async_copy(src_hbm.at[pl.ds(c * CH, CH)], sbuf, sem)

    def consume(dbuf, sbuf, sem):
        pltpu.make_async_copy(dst_hbm.at[pl.ds(0, CH)], dbuf, sem).wait()
        pltpu.make_async_copy(src_hbm.at[pl.ds(0, CH)], sbuf, sem).wait()

        def sub(i, kv):
            kv0, kv1 = kv
            d16 = dbuf[pl.ds(i * L, L)]
            s16 = sbuf[pl.ds(i * L, L)]
            off = d16 - base0
            m0 = (off >= 0) & (off < NPR)
            m1 = (off >= NPR) & (off < 2 * NPR)
            pos0 = kv0 + plsc.cumsum(m0.astype(jnp.int32)) - 1
            plsc.store_scatter(stg_s0, [pos0], s16, mask=m0)
            plsc.store_scatter(stg_o0, [pos0], off, mask=m0)
            pos1 = kv1 + plsc.cumsum(m1.astype(jnp.int32)) - 1
            plsc.store_scatter(stg_s1, [pos1], s16, mask=m1)
            plsc.store_scatter(stg_o1, [pos1], off - NPR, mask=m1)
            return (kv0 + plsc.all_reduce_population_count(m0),
                    kv1 + plsc.all_reduce_population_count(m1))

        k0 = jnp.full((L,), kref[0], jnp.int32)
        k1 = jnp.full((L,), kref[2], jnp.int32)
        kv0, kv1 = lax.fori_loop(0, CH // L, sub, (k0, k1))
        kref[0] = jnp.max(kv0)
        kref[2] = jnp.max(kv1)
        for r in range(2):
            for _ in range(NFLUSH):
                flush(r, kref[2 * r] >= FB)

    for p in range(2):
        if p < NCHUNK:
            start(p, *bufs[p])

    def pair(i2, _):
        for p in range(2):
            b = 2 * i2 + p
            dbuf, sbuf, sem = bufs[p]

            @pl.when(b < NCHUNK)
            def _(b=b, dbuf=dbuf, sbuf=sbuf, sem=sem):
                consume(dbuf, sbuf, sem)

                @pl.when(b + 2 < NCHUNK)
                def _():
                    start(b + 2, dbuf, sbuf, sem)
        return 0

    lax.fori_loop(0, (NCHUNK + 1) // 2, pair, 0)

    zeros = jnp.zeros((L,), jnp.int32)
    trash = jnp.full((L,), NPR, jnp.int32)
    iota = lax.iota(jnp.int32, L)
    for r in range(2):
        g = 2 * wid + r
        stg_s, stg_o = stg[r]
        k = kref[2 * r]
        cnt = kref[2 * r + 1] + k
        for i in range(FB // L):
            pos = k + i * L + iota
            plsc.store_scatter(stg_s, [pos], zeros)
            plsc.store_scatter(stg_o, [pos], trash)
        kref[2 * r] = k + FB
        flush(r, jnp.bool_(True))
        cbuf[...] = jnp.full((L,), cnt, jnp.int32)
        pltpu.sync_copy(cbuf, cnts_out.at[pl.ds(_al(g * L), L)])


def _make_sc_layer(with_deg):
    outs = [jax.ShapeDtypeStruct((NTOT * C,), jnp.float32)] * 4
    scratch = [
        pltpu.VMEM(((NPR + 1) * C,), jnp.float32),
        pltpu.VMEM(((NPR + 1) * C,), jnp.float32),
        pltpu.VMEM(((NPR + 1) * C,), jnp.float32),
        pltpu.VMEM(((NPR + 1) * C,), jnp.float32),
        pltpu.VMEM((FB, C), jnp.float32),
        pltpu.VMEM((FB, C), jnp.float32),
        pltpu.VMEM((FB,), jnp.int32),
        pltpu.VMEM((FB,), jnp.int32),
        pltpu.VMEM((FB + L,), jnp.int32),
        pltpu.VMEM((FB + L,), jnp.int32),
        pltpu.VMEM((L,), jnp.int32),
        pltpu.SemaphoreType.DMA,
        pltpu.SemaphoreType.DMA,
    ]
    if with_deg:
        outs = outs + [jax.ShapeDtypeStruct((NTOT * L,), jnp.float32)]
        scratch = scratch + [pltpu.VMEM(((NPR + 1) * L,), jnp.float32)]

    def body(a_hbm, srcs, offs, cnts, s1_o, s2_o, mx_o, mn_o, *rest):
        if with_deg:
            deg_o = rest[0]
            rest = rest[1:]
            accD = rest[-1]
            rest = rest[:-1]
        (accS, accQ, accMX, accMN, rows0, rows1, sbuf0, sbuf1,
         obuf0, obuf1, cbuf, sem0, sem1) = rest
        wid = _wid()
        zero = jnp.zeros((L,), jnp.float32)
        neg = jnp.full((L,), -3.4e38, jnp.float32)
        pos = jnp.full((L,), 3.4e38, jnp.float32)
        ones = jnp.ones((L,), jnp.float32)
        bufs = ((rows0, sbuf0, obuf0, sem0), (rows1, sbuf1, obuf1, sem1))

        for r in range(2):
            g = 2 * wid + r
            ebase = g * ECAP
            pltpu.sync_copy(cnts.at[pl.ds(_al(g * L), L)], cbuf)
            cnt = jnp.max(cbuf[...])
            nblk = (cnt + FB - 1) // FB

            def initrow(i, _):
                for j in range(C // L):
                    sl = pl.ds(i * C + j * L, L)
                    accS[sl] = zero
                    accQ[sl] = zero
                    accMX[sl] = neg
                    accMN[sl] = pos
                if with_deg:
                    accD[pl.ds(i * L, L)] = zero
                return 0

            lax.fori_loop(0, NPR, initrow, 0)

            def start(b, rows, sbuf, obuf, sem):
                pltpu.sync_copy(srcs.at[pl.ds(_al(ebase + b * FB), FB)], sbuf)
                pltpu.sync_copy(offs.at[pl.ds(_al(ebase + b * FB), FB)],
                                obuf.at[pl.ds(0, FB)])
                pltpu.async_copy(a_hbm.at[sbuf], rows, sem)

            def consume(rows, sbuf, obuf, sem):
                pltpu.make_async_copy(a_hbm.at[sbuf], rows, sem).wait()

                def edge(e, _):
                    off = _sload(obuf, e)
                    for j in range(C // L):
                        sl = pl.ds(off * C + j * L, L)
                        x = rows[e, pl.ds(j * L, L)]
                        accS[sl] = accS[sl] + x
                        accQ[sl] = accQ[sl] + x * x
                        accMX[sl] = jnp.maximum(accMX[sl], x)
                        accMN[sl] = jnp.minimum(accMN[sl], x)
                    if with_deg:
                        dsl = pl.ds(off * L, L)
                        accD[dsl] = accD[dsl] + ones
                    return 0

                lax.fori_loop(0, FB, edge, 0, unroll=4)

            for p in range(2):
                @pl.when(p < nblk)
                def _(p=p):
                    rows, sbuf, obuf, sem = bufs[p]
                    start(p, rows, sbuf, obuf, sem)

            def pair(i2, _):
                for p in range(2):
                    b = 2 * i2 + p
                    rows, sbuf, obuf, sem = bufs[p]

                    @pl.when(b < nblk)
                    def _(b=b, rows=rows, sbuf=sbuf, obuf=obuf, sem=sem):
                        consume(rows, sbuf, obuf, sem)

                        @pl.when(b + 2 < nblk)
                        def _():
                            start(b + 2, rows, sbuf, obuf, sem)
                return 0

            lax.fori_loop(0, (nblk + 1) // 2, pair, 0)

            hsl = pl.ds(_al(g * (NPR * C)), NPR * C)
            pltpu.sync_copy(accS.at[pl.ds(0, NPR * C)], s1_o.at[hsl])
            pltpu.sync_copy(accQ.at[pl.ds(0, NPR * C)], s2_o.at[hsl])
            pltpu.sync_copy(accMX.at[pl.ds(0, NPR * C)], mx_o.at[hsl])
            pltpu.sync_copy(accMN.at[pl.ds(0, NPR * C)], mn_o.at[hsl])
            if with_deg:
                pltpu.sync_copy(
                    accD.at[pl.ds(0, NPR * L)],
                    deg_o.at[pl.ds(_al(g * (NPR * L)), NPR * L)])

    return pl.kernel(
        body,
        out_type=tuple(outs),
        mesh=_mesh,
        compiler_params=_params,
        scratch_types=scratch,
    )


_sc_layer0 = _make_sc_layer(True)
_sc_layer = _make_sc_layer(False)


BLK = 400


def _pre_body(h_ref, w_ref, b_ref, a_ref, bout_ref):
    h = h_ref[...]
    a_ref[...] = jnp.dot(h, w_ref[0:C, :], preferred_element_type=jnp.float32)
    bout_ref[...] = (jnp.dot(h, w_ref[C:2 * C, :],
                             preferred_element_type=jnp.float32)
                     + b_ref[...])


_pre = pl.pallas_call(
    _pre_body,
    grid=(N // BLK,),
    in_specs=[
        pl.BlockSpec((BLK, C), lambda i: (i, 0)),
        pl.BlockSpec((2 * C, C), lambda i: (0, 0)),
        pl.BlockSpec((1, C), lambda i: (0, 0)),
    ],
    out_specs=[
        pl.BlockSpec((BLK, C), lambda i: (i, 0)),
        pl.BlockSpec((BLK, C), lambda i: (i, 0)),
    ],
    out_shape=[
        jax.ShapeDtypeStruct((N, C), jnp.float32),
        jax.ShapeDtypeStruct((N, C), jnp.float32),
    ],
)


def _post_body(residual, h_ref, b_ref, s1_ref, s2_ref, mx_ref, mn_ref,
               deg_ref, uw_ref, ub_ref, mw_ref, mb_ref, o_ref):
    h = h_ref[...]
    B = b_ref[...]
    d = deg_ref[:, 0:1]
    cnt = jnp.maximum(d, 1.0)
    has = d > 0
    S1 = s1_ref[...]
    mean = (S1 + d * B) / cnt
    msq = (s2_ref[...] + 2.0 * B * S1 + d * B * B) / cnt
    std = jnp.sqrt(jnp.maximum(msq - mean * mean, 0.0) + 1e-5)
    mx = jnp.where(has, mx_ref[...] + B, 0.0)
    mn = jnp.where(has, mn_ref[...] + B, 0.0)
    aggs = jnp.concatenate([mean, mx, mn, std], axis=1)
    logd = jnp.log(d + 1.0)
    amp = logd / DELTA
    att = jnp.where(logd > 0, DELTA / jnp.where(logd > 0, logd, 1.0), 0.0)
    uw = uw_ref[...]
    u = (jnp.dot(h, uw[0:C, :], preferred_element_type=jnp.float32)
         + jnp.dot(aggs, uw[C:C + 512, :], preferred_element_type=jnp.float32)
         + jnp.dot(aggs * amp, uw[C + 512:C + 1024, :],
                   preferred_element_type=jnp.float32)
         + jnp.dot(aggs * att, uw[C + 1024:C + 1536, :],
                   preferred_element_type=jnp.float32)
         + ub_ref[...])
    o = jnp.dot(u, mw_ref[...], preferred_element_type=jnp.float32) + mb_ref[...]
    o = jnp.where(o >= 0, o, 0.01 * o)
    if residual:
        o = o + h
    o_ref[...] = o


@functools.cache
def _post(cout, residual):
    uin = 13 * C
    full_spec = pl.BlockSpec((BLK, C), lambda i: (i, 0))
    return pl.pallas_call(
        functools.partial(_post_body, residual),
        grid=(N // BLK,),
        in_specs=[
            full_spec,
            full_spec,
            full_spec, full_spec, full_spec, full_spec,
            pl.BlockSpec((BLK, L), lambda i: (i, 0)),
            pl.BlockSpec((uin, cout), lambda i: (0, 0)),
            pl.BlockSpec((1, cout), lambda i: (0, 0)),
            pl.BlockSpec((cout, cout), lambda i: (0, 0)),
            pl.BlockSpec((1, cout), lambda i: (0, 0)),
        ],
        out_specs=pl.BlockSpec((BLK, cout), lambda i: (i, 0)),
        out_shape=jax.ShapeDtypeStruct((N, cout), jnp.float32),
    )


def _mean_body(x_ref, o_ref):
    o_ref[...] = jnp.mean(x_ref[...], axis=0, keepdims=True)


def _mean(x):
    n, c = x.shape
    return pl.pallas_call(
        _mean_body,
        out_shape=jax.ShapeDtypeStruct((1, c), jnp.float32),
    )(x)


def kernel(features, edge_index,
           M_W0, M_b0, U_W0, U_b0, mix_W0, mix_b0,
           M_W1, M_b1, U_W1, U_b1, mix_W1, mix_b1,
           M_W2, M_b2, U_W2, U_b2, mix_W2, mix_b2):
    src = edge_index[0]
    dst = edge_index[1]
    srcs, offs, cnts = _phase_a(src, dst)

    params = [
        (M_W0, M_b0, U_W0, U_b0, mix_W0, mix_b0),
        (M_W1, M_b1, U_W1, U_b1, mix_W1, mix_b1),
        (M_W2, M_b2, U_W2, U_b2, mix_W2, mix_b2),
    ]
    h = features
    deg = None
    for l, (MW, Mb, UW, Ub, mixW, mixb) in enumerate(params):
        a, B = _pre(h, MW, Mb.reshape(1, C))
        if l == 0:
            s1, s2, mx, mn, degf = _sc_layer0(a, srcs, offs, cnts)
            deg = degf.reshape(NTOT, L)[0:N]
            stats = (s1, s2, mx, mn)
        else:
            stats = _sc_layer(a, srcs, offs, cnts)
        s1, s2, mx, mn = (s.reshape(NTOT, C)[0:N] for s in stats)
        cout = UW.shape[1]
        h = _post(cout, l < 2)(h, B, s1, s2, mx, mn,
                               deg, UW, Ub.reshape(1, cout), mixW,
                               mixb.reshape(1, cout))
    return _mean(h)


# speedup vs baseline: 1.2031x; 1.0708x over previous
import functools

import jax
import jax.numpy as jnp
from jax import lax
from jax.experimental import pallas as pl
from jax.experimental.pallas import tpu as pltpu
from jax.experimental.pallas import tpu_sc as plsc

N = 10000
E = 320000
C = 128
DELTA = 3.5

NC, NS, L = 2, 16, 16
NW = NC * NS
NR = 2 * NW
NPR = 160
NTOT = NR * NPR
CH = 512
FB = 128
OB = 640
ECAP = E + FB
NFLUSH = 4

_mesh = plsc.VectorSubcoreMesh(core_axis_name="c", subcore_axis_name="s")
_params = pltpu.CompilerParams(needs_layout_passes=False)


def _wid():
    return lax.axis_index("s") * NC + lax.axis_index("c")


def _al(x):
    return pl.multiple_of(x, 8)


def _sload(ref, i):
    return ref[pl.ds(i, L)][0]


@functools.partial(
    pl.kernel,
    out_type=(
        jax.ShapeDtypeStruct((NR * ECAP,), jnp.int32),
        jax.ShapeDtypeStruct((NR * ECAP,), jnp.int32),
        jax.ShapeDtypeStruct((NR * L,), jnp.int32),
    ),
    mesh=_mesh,
    compiler_params=_params,
    scratch_types=[
        pltpu.VMEM((CH,), jnp.int32),
        pltpu.VMEM((CH,), jnp.int32),
        pltpu.VMEM((CH,), jnp.int32),
        pltpu.VMEM((CH,), jnp.int32),
        pltpu.VMEM((OB,), jnp.int32),
        pltpu.VMEM((OB,), jnp.int32),
        pltpu.VMEM((OB,), jnp.int32),
        pltpu.VMEM((OB,), jnp.int32),
        pltpu.VMEM((L,), jnp.int32),
        pltpu.SMEM((8,), jnp.int32),
        pltpu.SemaphoreType.DMA,
        pltpu.SemaphoreType.DMA,
    ],
)
def _phase_a(src_hbm, dst_hbm, srcs_out, offs_out, cnts_out,
             dbuf0, dbuf1, sbuf0, sbuf1, stg_s0, stg_o0, stg_s1, stg_o1,
             cbuf, kref, sem0, sem1):
    wid = _wid()
    base0 = (2 * wid) * NPR
    stg = ((stg_s0, stg_o0), (stg_s1, stg_o1))
    bufs = ((dbuf0, sbuf0, sem0), (dbuf1, sbuf1, sem1))
    NCHUNK = E // CH

    kref[0] = 0
    kref[1] = 0
    kref[2] = 0
    kref[3] = 0

    def flush(r, pred):
        stg_s, stg_o = stg[r]
        ebase = (2 * wid + r) * ECAP

        @pl.when(pred)
        def _():
            ptr = kref[2 * r + 1]
            pltpu.sync_copy(stg_s.at[pl.ds(0, FB)],
                            srcs_out.at[pl.ds(_al(ebase + ptr), FB)])
            pltpu.sync_copy(stg_o.at[pl.ds(0, FB)],
                            offs_out.at[pl.ds(_al(ebase + ptr), FB)])

            def mv(i, _):
                stg_s[pl.ds(i * L, L)] = stg_s[pl.ds(i * L + FB, L)]
                stg_o[pl.ds(i * L, L)] = stg_o[pl.ds(i * L + FB, L)]
                return 0

            lax.fori_loop(0, (OB - FB) // L, mv, 0)
            kref[2 * r] = kref[2 * r] - FB
            kref[2 * r + 1] = ptr + FB

    def start(c, dbuf, sbuf, sem):
        pltpu.async_copy(dst_hbm.at[pl.ds(c * CH, CH)], dbuf, sem)
        pltpu.async_copy(src_hbm.at[pl.ds(c * CH, CH)], sbuf, sem)

    def consume(dbuf, sbuf, sem):
        pltpu.make_async_copy(dst_hbm.at[pl.ds(0, CH)], dbuf, sem).wait()
        pltpu.make_async_copy(src_hbm.at[pl.ds(0, CH)], sbuf, sem).wait()

        def sub(i, kv):
            kv0, kv1 = kv
            d16 = dbuf[pl.ds(i * L, L)]
            s16 = sbuf[pl.ds(i * L, L)]
            off = d16 - base0
            m0 = (off >= 0) & (off < NPR)
            m1 = (off >= NPR) & (off < 2 * NPR)
            pos0 = kv0 + plsc.cumsum(m0.astype(jnp.int32)) - 1
            plsc.store_scatter(stg_s0, [pos0], s16, mask=m0)
            plsc.store_scatter(stg_o0, [pos0], off, mask=m0)
            pos1 = kv1 + plsc.cumsum(m1.astype(jnp.int32)) - 1
            plsc.store_scatter(stg_s1, [pos1], s16, mask=m1)
            plsc.store_scatter(stg_o1, [pos1], off - NPR, mask=m1)
            return (kv0 + plsc.all_reduce_population_count(m0),
                    kv1 + plsc.all_reduce_population_count(m1))

        k0 = jnp.full((L,), kref[0], jnp.int32)
        k1 = jnp.full((L,), kref[2], jnp.int32)
        kv0, kv1 = lax.fori_loop(0, CH // L, sub, (k0, k1))
        kref[0] = jnp.max(kv0)
        kref[2] = jnp.max(kv1)
        for r in range(2):
            for _ in range(NFLUSH):
                flush(r, kref[2 * r] >= FB)

    for p in range(2):
        if p < NCHUNK:
            start(p, *bufs[p])

    def pair(i2, _):
        for p in range(2):
            b = 2 * i2 + p
            dbuf, sbuf, sem = bufs[p]

            @pl.when(b < NCHUNK)
            def _(b=b, dbuf=dbuf, sbuf=sbuf, sem=sem):
                consume(dbuf, sbuf, sem)

                @pl.when(b + 2 < NCHUNK)
                def _():
                    start(b + 2, dbuf, sbuf, sem)
        return 0

    lax.fori_loop(0, (NCHUNK + 1) // 2, pair, 0)

    zeros = jnp.zeros((L,), jnp.int32)
    trash = jnp.full((L,), NPR, jnp.int32)
    iota = lax.iota(jnp.int32, L)
    for r in range(2):
        g = 2 * wid + r
        stg_s, stg_o = stg[r]
        k = kref[2 * r]
        cnt = kref[2 * r + 1] + k
        for i in range(FB // L):
            pos = k + i * L + iota
            plsc.store_scatter(stg_s, [pos], zeros)
            plsc.store_scatter(stg_o, [pos], trash)
        kref[2 * r] = k + FB
        flush(r, jnp.bool_(True))
        cbuf[...] = jnp.full((L,), cnt, jnp.int32)
        pltpu.sync_copy(cbuf, cnts_out.at[pl.ds(_al(g * L), L)])


def _make_sc_layer(with_deg):
    outs = [jax.ShapeDtypeStruct((NTOT * C,), jnp.float32)] * 4
    scratch = [
        pltpu.VMEM(((NPR + 1) * C,), jnp.float32),
        pltpu.VMEM(((NPR + 1) * C,), jnp.float32),
        pltpu.VMEM(((NPR + 1) * C,), jnp.float32),
        pltpu.VMEM(((NPR + 1) * C,), jnp.float32),
        pltpu.VMEM((FB, C), jnp.float32),
        pltpu.VMEM((FB, C), jnp.float32),
        pltpu.VMEM((FB,), jnp.int32),
        pltpu.VMEM((FB,), jnp.int32),
        pltpu.VMEM((FB,), jnp.int32),
        pltpu.VMEM((FB,), jnp.int32),
        pltpu.VMEM((FB + L,), jnp.int32),
        pltpu.VMEM((FB + L,), jnp.int32),
        pltpu.VMEM((FB + L,), jnp.int32),
        pltpu.VMEM((FB + L,), jnp.int32),
        pltpu.VMEM((L,), jnp.int32),
        pltpu.SemaphoreType.DMA,
        pltpu.SemaphoreType.DMA,
        pltpu.SemaphoreType.DMA,
        pltpu.SemaphoreType.DMA,
        pltpu.SemaphoreType.DMA,
        pltpu.SemaphoreType.DMA,
    ]
    if with_deg:
        outs = outs + [jax.ShapeDtypeStruct((NTOT * L,), jnp.float32)]
        scratch = scratch + [pltpu.VMEM(((NPR + 1) * L,), jnp.float32)]

    def body(a_hbm, srcs, offs, cnts, s1_o, s2_o, mx_o, mn_o, *rest):
        if with_deg:
            deg_o = rest[0]
            rest = rest[1:]
            accD = rest[-1]
            rest = rest[:-1]
        (accS, accQ, accMX, accMN, rows0, rows1, sb0, sb1, sb2, sb3,
         ob0, ob1, ob2, ob3, cbuf, gsem0, gsem1,
         isem0, isem1, isem2, isem3) = rest
        wid = _wid()
        zero = jnp.zeros((L,), jnp.float32)
        neg = jnp.full((L,), -3.4e38, jnp.float32)
        pos = jnp.full((L,), 3.4e38, jnp.float32)
        ones = jnp.ones((L,), jnp.float32)
        rows = (rows0, rows1)
        gsem = (gsem0, gsem1)
        sb = (sb0, sb1, sb2, sb3)
        ob = (ob0, ob1, ob2, ob3)
        isem = (isem0, isem1, isem2, isem3)

        for r in range(2):
            g = 2 * wid + r
            ebase = g * ECAP
            pltpu.sync_copy(cnts.at[pl.ds(_al(g * L), L)], cbuf)
            cnt = jnp.max(cbuf[...])
            nblk = (cnt + FB - 1) // FB

            def initrow(i, _):
                for j in range(C // L):
                    sl = pl.ds(i * C + j * L, L)
                    accS[sl] = zero
                    accQ[sl] = zero
                    accMX[sl] = neg
                    accMN[sl] = pos
                if with_deg:
                    accD[pl.ds(i * L, L)] = zero
                return 0

            lax.fori_loop(0, NPR, initrow, 0)

            def start_idx(b, q):
                pltpu.async_copy(srcs.at[pl.ds(_al(ebase + b * FB), FB)],
                                 sb[q], isem[q])
                pltpu.async_copy(offs.at[pl.ds(_al(ebase + b * FB), FB)],
                                 ob[q].at[pl.ds(0, FB)], isem[q])

            def fire_gather(q, v):
                pltpu.make_async_copy(srcs.at[pl.ds(0, FB)],
                                      sb[q], isem[q]).wait()
                pltpu.make_async_copy(srcs.at[pl.ds(0, FB)],
                                      ob[q].at[pl.ds(0, FB)], isem[q]).wait()
                pltpu.async_copy(a_hbm.at[sb[q]], rows[v], gsem[v])

            def edges(v, q):
                pltpu.make_async_copy(a_hbm.at[sb[q]], rows[v],
                                      gsem[v]).wait()
                obuf = ob[q]
                rws = rows[v]

                def edge(e, _):
                    off = _sload(obuf, e)
                    for j in range(C // L):
                        sl = pl.ds(off * C + j * L, L)
                        x = rws[e, pl.ds(j * L, L)]
                        accS[sl] = accS[sl] + x
                        accQ[sl] = accQ[sl] + x * x
                        accMX[sl] = jnp.maximum(accMX[sl], x)
                        accMN[sl] = jnp.minimum(accMN[sl], x)
                    if with_deg:
                        dsl = pl.ds(off * L, L)
                        accD[dsl] = accD[dsl] + ones
                    return 0

                lax.fori_loop(0, FB, edge, 0, unroll=2)

            for p in range(3):
                @pl.when(p < nblk)
                def _(p=p):
                    start_idx(p, p)

            @pl.when(0 < nblk)
            def _():
                fire_gather(0, 0)

            def quad(i4, _):
                for p in range(4):
                    b = 4 * i4 + p

                    @pl.when(b < nblk)
                    def _(b=b, p=p):
                        @pl.when(b + 1 < nblk)
                        def _():
                            fire_gather((p + 1) % 4, (p + 1) % 2)

                        @pl.when(b + 3 < nblk)
                        def _():
                            start_idx(b + 3, (p + 3) % 4)

                        edges(p % 2, p)
                return 0

            lax.fori_loop(0, (nblk + 3) // 4, quad, 0)

            hsl = pl.ds(_al(g * (NPR * C)), NPR * C)
            pltpu.sync_copy(accS.at[pl.ds(0, NPR * C)], s1_o.at[hsl])
            pltpu.sync_copy(accQ.at[pl.ds(0, NPR * C)], s2_o.at[hsl])
            pltpu.sync_copy(accMX.at[pl.ds(0, NPR * C)], mx_o.at[hsl])
            pltpu.sync_copy(accMN.at[pl.ds(0, NPR * C)], mn_o.at[hsl])
            if with_deg:
                pltpu.sync_copy(
                    accD.at[pl.ds(0, NPR * L)],
                    deg_o.at[pl.ds(_al(g * (NPR * L)), NPR * L)])

    return pl.kernel(
        body,
        out_type=tuple(outs),
        mesh=_mesh,
        compiler_params=_params,
        scratch_types=scratch,
    )


_sc_layer0 = _make_sc_layer(True)
_sc_layer = _make_sc_layer(False)


BLK = 400


def _pre_body(h_ref, w_ref, b_ref, a_ref, bout_ref):
    h = h_ref[...]
    a_ref[...] = jnp.dot(h, w_ref[0:C, :], preferred_element_type=jnp.float32)
    bout_ref[...] = (jnp.dot(h, w_ref[C:2 * C, :],
                             preferred_element_type=jnp.float32)
                     + b_ref[...])


_pre = pl.pallas_call(
    _pre_body,
    grid=(N // BLK,),
    in_specs=[
        pl.BlockSpec((BLK, C), lambda i: (i, 0)),
        pl.BlockSpec((2 * C, C), lambda i: (0, 0)),
        pl.BlockSpec((1, C), lambda i: (0, 0)),
    ],
    out_specs=[
        pl.BlockSpec((BLK, C), lambda i: (i, 0)),
        pl.BlockSpec((BLK, C), lambda i: (i, 0)),
    ],
    out_shape=[
        jax.ShapeDtypeStruct((N, C), jnp.float32),
        jax.ShapeDtypeStruct((N, C), jnp.float32),
    ],
)


def _post_body(residual, h_ref, b_ref, s1_ref, s2_ref, mx_ref, mn_ref,
               deg_ref, uw_ref, ub_ref, mw_ref, mb_ref, o_ref):
    h = h_ref[...]
    B = b_ref[...]
    d = deg_ref[:, 0:1]
    cnt = jnp.maximum(d, 1.0)
    has = d > 0
    S1 = s1_ref[...]
    mean = (S1 + d * B) / cnt
    msq = (s2_ref[...] + 2.0 * B * S1 + d * B * B) / cnt
    std = jnp.sqrt(jnp.maximum(msq - mean * mean, 0.0) + 1e-5)
    mx = jnp.where(has, mx_ref[...] + B, 0.0)
    mn = jnp.where(has, mn_ref[...] + B, 0.0)
    aggs = jnp.concatenate([mean, mx, mn, std], axis=1)
    logd = jnp.log(d + 1.0)
    amp = logd / DELTA
    att = jnp.where(logd > 0, DELTA / jnp.where(logd > 0, logd, 1.0), 0.0)
    uw = uw_ref[...]
    u = (jnp.dot(h, uw[0:C, :], preferred_element_type=jnp.float32)
         + jnp.dot(aggs, uw[C:C + 512, :], preferred_element_type=jnp.float32)
         + jnp.dot(aggs * amp, uw[C + 512:C + 1024, :],
                   preferred_element_type=jnp.float32)
         + jnp.dot(aggs * att, uw[C + 1024:C + 1536, :],
                   preferred_element_type=jnp.float32)
         + ub_ref[...])
    o = jnp.dot(u, mw_ref[...], preferred_element_type=jnp.float32) + mb_ref[...]
    o = jnp.where(o >= 0, o, 0.01 * o)
    if residual:
        o = o + h
    o_ref[...] = o


@functools.cache
def _post(cout, residual):
    uin = 13 * C
    full_spec = pl.BlockSpec((BLK, C), lambda i: (i, 0))
    return pl.pallas_call(
        functools.partial(_post_body, residual),
        grid=(N // BLK,),
        in_specs=[
            full_spec,
            full_spec,
            full_spec, full_spec, full_spec, full_spec,
            pl.BlockSpec((BLK, L), lambda i: (i, 0)),
            pl.BlockSpec((uin, cout), lambda i: (0, 0)),
            pl.BlockSpec((1, cout), lambda i: (0, 0)),
            pl.BlockSpec((cout, cout), lambda i: (0, 0)),
            pl.BlockSpec((1, cout), lambda i: (0, 0)),
        ],
        out_specs=pl.BlockSpec((BLK, cout), lambda i: (i, 0)),
        out_shape=jax.ShapeDtypeStruct((N, cout), jnp.float32),
    )


def _mean_body(x_ref, o_ref):
    o_ref[...] = jnp.mean(x_ref[...], axis=0, keepdims=True)


def _mean(x):
    n, c = x.shape
    return pl.pallas_call(
        _mean_body,
        out_shape=jax.ShapeDtypeStruct((1, c), jnp.float32),
    )(x)


def kernel(features, edge_index,
           M_W0, M_b0, U_W0, U_b0, mix_W0, mix_b0,
           M_W1, M_b1, U_W1, U_b1, mix_W1, mix_b1,
           M_W2, M_b2, U_W2, U_b2, mix_W2, mix_b2):
    src = edge_index[0]
    dst = edge_index[1]
    srcs, offs, cnts = _phase_a(src, dst)

    params = [
        (M_W0, M_b0, U_W0, U_b0, mix_W0, mix_b0),
        (M_W1, M_b1, U_W1, U_b1, mix_W1, mix_b1),
        (M_W2, M_b2, U_W2, U_b2, mix_W2, mix_b2),
    ]
    h = features
    deg = None
    for l, (MW, Mb, UW, Ub, mixW, mixb) in enumerate(params):
        a, B = _pre(h, MW, Mb.reshape(1, C))
        if l == 0:
            s1, s2, mx, mn, degf = _sc_layer0(a, srcs, offs, cnts)
            deg = degf.reshape(NTOT, L)[0:N]
            stats = (s1, s2, mx, mn)
        else:
            stats = _sc_layer(a, srcs, offs, cnts)
        s1, s2, mx, mn = (s.reshape(NTOT, C)[0:N] for s in stats)
        cout = UW.shape[1]
        h = _post(cout, l < 2)(h, B, s1, s2, mx, mn,
                               deg, UW, Ub.reshape(1, cout), mixW,
                               mixb.reshape(1, cout))
    return _mean(h)
